# Initial kernel scaffold; baseline (speedup 1.0000x reference)
import jax
import jax.numpy as jnp
from jax.experimental import pallas as pl


def kernel(xyz, points, W0, b0, g0, be0, W1, b1, g1, be1, W2, b2, g2, be2):
    raise NotImplementedError("write your pallas kernel here")


# speedup vs baseline: 2.5374x; 2.5374x over previous
import jax
import jax.numpy as jnp
from jax.experimental import pallas as pl
from jax.experimental.pallas import tpu as pltpu

_NPOINT = 512
_NSAMPLE = 32


def _fps_body(xyz_ref, new_ref):
    x = xyz_ref[:, 0, :]
    y = xyz_ref[:, 1, :]
    z = xyz_ref[:, 2, :]
    B, N = x.shape
    S = new_ref.shape[2]
    iota = jax.lax.broadcasted_iota(jnp.int32, (B, N), 1)
    iota_s = jax.lax.broadcasted_iota(jnp.int32, (B, S), 1)

    def step(t, carry):
        dist, far, ax, ay, az = carry
        oh = iota == far
        cx = jnp.sum(jnp.where(oh, x, 0.0), axis=1, keepdims=True)
        cy = jnp.sum(jnp.where(oh, y, 0.0), axis=1, keepdims=True)
        cz = jnp.sum(jnp.where(oh, z, 0.0), axis=1, keepdims=True)
        sl = iota_s == t
        ax = jnp.where(sl, cx, ax)
        ay = jnp.where(sl, cy, ay)
        az = jnp.where(sl, cz, az)
        dx = x - cx
        dy = y - cy
        dz = z - cz
        d = (dx * dx + dz * dz) + dy * dy
        dist = jnp.minimum(dist, d)
        m = jnp.max(dist, axis=1, keepdims=True)
        far = jnp.min(jnp.where(dist == m, iota, N), axis=1, keepdims=True)
        return dist, far, ax, ay, az

    zs = jnp.zeros((B, S), jnp.float32)
    init = (jnp.full((B, N), 1e10, jnp.float32), jnp.zeros((B, 1), jnp.int32),
            zs, zs, zs)
    _, _, ax, ay, az = jax.lax.fori_loop(0, _NPOINT, step, init)
    new_ref[:, 0:1, :] = ax[:, None, :]
    new_ref[:, 1:2, :] = ay[:, None, :]
    new_ref[:, 2:3, :] = az[:, None, :]


def _knn_body(nxyz_ref, xyz_ref, idx_ref, d_ref):
    a = nxyz_ref[0]
    p = xyz_ref[0]
    TS = a.shape[1]
    N = p.shape[1]
    at = jnp.transpose(a)
    asq = jnp.sum(at * at, axis=1, keepdims=True)
    psq = jnp.sum(p * p, axis=0, keepdims=True)
    m3 = jax.lax.dot_general(
        at, p, (((1,), (0,)), ((), ())), preferred_element_type=jnp.float32)
    d_ref[...] = (-2.0 * m3 + asq) + psq
    iota = jax.lax.broadcasted_iota(jnp.int32, (TS, N), 1)
    K = idx_ref.shape[2]
    iota_k = jax.lax.broadcasted_iota(jnp.int32, (TS, K), 1)

    def step(k, acc):
        d = d_ref[...]
        m = jnp.min(d, axis=1, keepdims=True)
        sel = jnp.min(jnp.where(d == m, iota, N), axis=1, keepdims=True)
        acc = jnp.where(iota_k == k, sel, acc)
        d_ref[...] = jnp.where(iota == sel, jnp.float32(jnp.inf), d)
        return acc

    acc = jax.lax.fori_loop(0, _NSAMPLE, step, jnp.zeros((TS, K), jnp.int32))
    idx_ref[0] = acc


def _conv_bn_relu(x, W, b, g, be):
    y = jnp.einsum('bcks,oc->boks', x, W) + b[None, :, None, None]
    mean = jnp.mean(y, axis=(0, 2, 3), keepdims=True)
    var = jnp.mean((y - mean) ** 2, axis=(0, 2, 3), keepdims=True)
    y = (y - mean) / jnp.sqrt(var + 1e-5)
    y = y * g[None, :, None, None] + be[None, :, None, None]
    return jax.nn.relu(y)


def kernel(xyz, points, W0, b0, g0, be0, W1, b1, g1, be1, W2, b2, g2, be2):
    B, _, N = xyz.shape
    D = points.shape[1]
    S, K = _NPOINT, _NSAMPLE
    TS = 128

    new_xyz = pl.pallas_call(
        _fps_body,
        out_shape=jax.ShapeDtypeStruct((B, 3, S), jnp.float32),
    )(xyz)

    idx = pl.pallas_call(
        _knn_body,
        grid=(B, S // TS),
        in_specs=[
            pl.BlockSpec((1, 3, TS), lambda b, s: (b, 0, s)),
            pl.BlockSpec((1, 3, N), lambda b, s: (b, 0, 0)),
        ],
        out_specs=pl.BlockSpec((1, TS, K), lambda b, s: (b, s, 0)),
        out_shape=jax.ShapeDtypeStruct((B, S, K), jnp.int32),
        scratch_shapes=[pltpu.VMEM((TS, N), jnp.float32)],
    )(new_xyz, xyz)

    xyz_t = jnp.transpose(xyz, (0, 2, 1))
    pts_t = jnp.transpose(points, (0, 2, 1))
    new_xyz_t = jnp.transpose(new_xyz, (0, 2, 1))

    idx_e3 = jnp.broadcast_to(idx[:, :, :, None], (B, S, K, 3))
    grouped_xyz = jnp.take_along_axis(
        jnp.broadcast_to(xyz_t[:, None, :, :], (B, S, N, 3)), idx_e3, axis=2)
    grouped_xyz_norm = grouped_xyz - new_xyz_t[:, :, None, :]
    idx_eD = jnp.broadcast_to(idx[:, :, :, None], (B, S, K, D))
    grouped_points = jnp.take_along_axis(
        jnp.broadcast_to(pts_t[:, None, :, :], (B, S, N, D)), idx_eD, axis=2)
    feats = jnp.concatenate([grouped_xyz_norm, grouped_points], axis=-1)
    feats = jnp.transpose(feats, (0, 3, 2, 1))
    feats = _conv_bn_relu(feats, W0, b0, g0, be0)
    feats = _conv_bn_relu(feats, W1, b1, g1, be1)
    feats = _conv_bn_relu(feats, W2, b2, g2, be2)
    return (new_xyz, feats)


# speedup vs baseline: 10.8541x; 4.2776x over previous
import functools

import jax
import jax.numpy as jnp
from jax import lax
from jax.experimental import pallas as pl
from jax.experimental.pallas import tpu as pltpu
from jax.experimental.pallas import tpu_sc as plsc

_NPOINT = 512
_NSAMPLE = 32


def _fps_body(xyz_ref, new_ref):
    x = xyz_ref[:, 0, :]
    y = xyz_ref[:, 1, :]
    z = xyz_ref[:, 2, :]
    B, N = x.shape
    S = new_ref.shape[2]
    iota = jax.lax.broadcasted_iota(jnp.int32, (B, N), 1)
    iota_s = jax.lax.broadcasted_iota(jnp.int32, (B, S), 1)

    def step(t, carry):
        dist, far, ax, ay, az = carry
        oh = iota == far
        cx = jnp.sum(jnp.where(oh, x, 0.0), axis=1, keepdims=True)
        cy = jnp.sum(jnp.where(oh, y, 0.0), axis=1, keepdims=True)
        cz = jnp.sum(jnp.where(oh, z, 0.0), axis=1, keepdims=True)
        sl = iota_s == t
        ax = jnp.where(sl, cx, ax)
        ay = jnp.where(sl, cy, ay)
        az = jnp.where(sl, cz, az)
        dx = x - cx
        dy = y - cy
        dz = z - cz
        d = (dx * dx + dz * dz) + dy * dy
        dist = jnp.minimum(dist, d)
        m = jnp.max(dist, axis=1, keepdims=True)
        far = jnp.min(jnp.where(dist == m, iota, N), axis=1, keepdims=True)
        return dist, far, ax, ay, az

    zs = jnp.zeros((B, S), jnp.float32)
    init = (jnp.full((B, N), 1e10, jnp.float32), jnp.zeros((B, 1), jnp.int32),
            zs, zs, zs)
    _, _, ax, ay, az = jax.lax.fori_loop(0, _NPOINT, step, init)
    new_ref[:, 0:1, :] = ax[:, None, :]
    new_ref[:, 1:2, :] = ay[:, None, :]
    new_ref[:, 2:3, :] = az[:, None, :]


def _knn_body(nxyz_ref, xyz_ref, idx_ref, d_ref):
    a = nxyz_ref[0]
    p = xyz_ref[0]
    TS = a.shape[1]
    N = p.shape[1]
    b = pl.program_id(0)
    at = jnp.transpose(a)
    asq = jnp.sum(at * at, axis=1, keepdims=True)
    psq = jnp.sum(p * p, axis=0, keepdims=True)
    m3 = jax.lax.dot_general(
        at, p, (((1,), (0,)), ((), ())), preferred_element_type=jnp.float32)
    d_ref[...] = (-2.0 * m3 + asq) + psq
    iota = jax.lax.broadcasted_iota(jnp.int32, (TS, N), 1)
    K = idx_ref.shape[2]
    iota_k = jax.lax.broadcasted_iota(jnp.int32, (TS, K), 1)

    def step(k, acc):
        d = d_ref[...]
        m = jnp.min(d, axis=1, keepdims=True)
        sel = jnp.min(jnp.where(d == m, iota, N), axis=1, keepdims=True)
        acc = jnp.where(iota_k == k, sel, acc)
        d_ref[...] = jnp.where(iota == sel, jnp.float32(jnp.inf), d)
        return acc

    acc = jax.lax.fori_loop(0, _NSAMPLE, step, jnp.zeros((TS, K), jnp.int32))
    idx_ref[0] = acc + b * N


def _conv1_body(x_ref, nf_ref, w_ref, b_ref, y_ref, st_ref):
    i = pl.program_id(0)
    x = x_ref[...]
    nf = nf_ref[...]
    MT = x.shape[0]
    SR = nf.shape[0]
    w3 = w_ref[0:3, :]
    t1 = jax.lax.dot_general(
        nf, w3, (((1,), (0,)), ((), ())), preferred_element_type=jnp.float32)
    t1r = jnp.broadcast_to(t1[:, None, :], (SR, MT // SR, t1.shape[1]))
    t1r = t1r.reshape(MT, t1.shape[1])
    y = jax.lax.dot_general(
        x, w_ref[...], (((1,), (0,)), ((), ())),
        preferred_element_type=jnp.float32)
    y = (y - t1r) + b_ref[...]
    y_ref[...] = y

    @pl.when(i == 0)
    def _():
        st_ref[...] = jnp.zeros_like(st_ref)

    st_ref[0:1, :] += jnp.sum(y, axis=0, keepdims=True)
    st_ref[1:2, :] += jnp.sum(y * y, axis=0, keepdims=True)


def _make_bn_conv_body(m_total):
    inv_m = 1.0 / m_total

    def body(y_ref, st_in_ref, g_ref, be_ref, w_ref, b_ref, y2_ref, st_ref):
        i = pl.program_id(0)
        y = y_ref[...]
        st = st_in_ref[...]
        mean = st[0:1, :] * inv_m
        var = st[1:2, :] * inv_m - mean * mean
        yn = (y - mean) / jnp.sqrt(var + 1e-5)
        yn = yn * g_ref[...] + be_ref[...]
        xr = jnp.maximum(yn, 0.0)
        y2 = jax.lax.dot_general(
            xr, w_ref[...], (((1,), (0,)), ((), ())),
            preferred_element_type=jnp.float32) + b_ref[...]
        y2_ref[...] = y2

        @pl.when(i == 0)
        def _():
            st_ref[...] = jnp.zeros_like(st_ref)

        st_ref[0:1, :] += jnp.sum(y2, axis=0, keepdims=True)
        st_ref[1:2, :] += jnp.sum(y2 * y2, axis=0, keepdims=True)

    return body


def _make_bn_relu_body(m_total):
    inv_m = 1.0 / m_total

    def body(y_ref, st_in_ref, g_ref, be_ref, out_ref):
        y = y_ref[...]
        st = st_in_ref[...]
        mean = st[0:1, :] * inv_m
        var = st[1:2, :] * inv_m - mean * mean
        yn = (y - mean) / jnp.sqrt(var + 1e-5)
        yn = yn * g_ref[...] + be_ref[...]
        out_ref[...] = jnp.maximum(yn, 0.0)

    return body


def kernel(xyz, points, W0, b0, g0, be0, W1, b1, g1, be1, W2, b2, g2, be2):
    B, _, N = xyz.shape
    D = points.shape[1]
    S, K = _NPOINT, _NSAMPLE
    TS = 128
    M = B * S * K
    TBL_W = 128

    new_xyz = pl.pallas_call(
        _fps_body,
        out_shape=jax.ShapeDtypeStruct((B, 3, S), jnp.float32),
    )(xyz)

    idx = pl.pallas_call(
        _knn_body,
        grid=(B, S // TS),
        in_specs=[
            pl.BlockSpec((1, 3, TS), lambda b, s: (b, 0, s)),
            pl.BlockSpec((1, 3, N), lambda b, s: (b, 0, 0)),
        ],
        out_specs=pl.BlockSpec((1, TS, K), lambda b, s: (b, s, 0)),
        out_shape=jax.ShapeDtypeStruct((B, S, K), jnp.int32),
        scratch_shapes=[pltpu.VMEM((TS, N), jnp.float32)],
    )(new_xyz, xyz)

    xyz_rows = jnp.transpose(xyz, (0, 2, 1)).reshape(B * N, 3)
    pts_rows = jnp.transpose(points, (0, 2, 1)).reshape(B * N, D)
    tbl = jnp.concatenate(
        [xyz_rows, pts_rows,
         jnp.zeros((B * N, TBL_W - 3 - D), jnp.float32)], axis=1)
    idx2d = idx.reshape(M // 128, 128)

    NW = 32
    rows_per_w = M // NW
    idx_rows_per_w = rows_per_w // 128
    mesh = plsc.VectorSubcoreMesh(core_axis_name="c", subcore_axis_name="s")

    @functools.partial(
        pl.kernel, mesh=mesh,
        out_type=jax.ShapeDtypeStruct((M, TBL_W), jnp.float32),
        scratch_types=[
            pltpu.VMEM((idx_rows_per_w, 128), jnp.int32),
            pltpu.VMEM((128, TBL_W), jnp.float32),
            pltpu.SemaphoreType.DMA,
        ],
    )
    def _gather_sc(tbl_hbm, idx_hbm, out_hbm, idx_v, rows_v, sem):
        wid = lax.axis_index("s") * 2 + lax.axis_index("c")
        pltpu.sync_copy(
            idx_hbm.at[pl.ds(wid * idx_rows_per_w, idx_rows_per_w)], idx_v)

        def body(j, _):
            pltpu.async_copy(tbl_hbm.at[idx_v.at[j]], rows_v, sem).wait()
            pltpu.sync_copy(
                rows_v, out_hbm.at[pl.ds(wid * rows_per_w + j * 128, 128)])
            return 0

        lax.fori_loop(0, idx_rows_per_w, body, 0)

    x0 = _gather_sc(tbl, idx2d)

    new_flat = jnp.transpose(new_xyz, (0, 2, 1)).reshape(B * S, 3)
    w0p = jnp.concatenate(
        [jnp.transpose(W0), jnp.zeros((TBL_W - 3 - D, W0.shape[0]),
                                      jnp.float32)], axis=0)
    MT = 4096
    grid1 = (M // MT,)
    C1 = W0.shape[0]
    y0, st0 = pl.pallas_call(
        _conv1_body,
        grid=grid1,
        in_specs=[
            pl.BlockSpec((MT, TBL_W), lambda i: (i, 0)),
            pl.BlockSpec((MT // K, 3), lambda i: (i, 0)),
            pl.BlockSpec((TBL_W, C1), lambda i: (0, 0)),
            pl.BlockSpec((1, C1), lambda i: (0, 0)),
        ],
        out_specs=[
            pl.BlockSpec((MT, C1), lambda i: (i, 0)),
            pl.BlockSpec((2, C1), lambda i: (0, 0)),
        ],
        out_shape=[
            jax.ShapeDtypeStruct((M, C1), jnp.float32),
            jax.ShapeDtypeStruct((2, C1), jnp.float32),
        ],
    )(x0, new_flat, w0p, b0[None, :])

    def bn_conv(y, st, g, be, wT, b, c_out):
        c_in = y.shape[1]
        return pl.pallas_call(
            _make_bn_conv_body(float(M)),
            grid=grid1,
            in_specs=[
                pl.BlockSpec((MT, c_in), lambda i: (i, 0)),
                pl.BlockSpec((2, c_in), lambda i: (0, 0)),
                pl.BlockSpec((1, c_in), lambda i: (0, 0)),
                pl.BlockSpec((1, c_in), lambda i: (0, 0)),
                pl.BlockSpec((c_in, c_out), lambda i: (0, 0)),
                pl.BlockSpec((1, c_out), lambda i: (0, 0)),
            ],
            out_specs=[
                pl.BlockSpec((MT, c_out), lambda i: (i, 0)),
                pl.BlockSpec((2, c_out), lambda i: (0, 0)),
            ],
            out_shape=[
                jax.ShapeDtypeStruct((M, c_out), jnp.float32),
                jax.ShapeDtypeStruct((2, c_out), jnp.float32),
            ],
        )(y, st, g[None, :], be[None, :], wT, b[None, :])

    y1, st1 = bn_conv(y0, st0, g0, be0, jnp.transpose(W1), b1, W1.shape[0])
    y2, st2 = bn_conv(y1, st1, g1, be1, jnp.transpose(W2), b2, W2.shape[0])

    C3 = W2.shape[0]
    out_flat = pl.pallas_call(
        _make_bn_relu_body(float(M)),
        grid=grid1,
        in_specs=[
            pl.BlockSpec((MT, C3), lambda i: (i, 0)),
            pl.BlockSpec((2, C3), lambda i: (0, 0)),
            pl.BlockSpec((1, C3), lambda i: (0, 0)),
            pl.BlockSpec((1, C3), lambda i: (0, 0)),
        ],
        out_specs=pl.BlockSpec((MT, C3), lambda i: (i, 0)),
        out_shape=jax.ShapeDtypeStruct((M, C3), jnp.float32),
    )(y2, st2, g2[None, :], be2[None, :])

    feats = jnp.transpose(out_flat.reshape(B, S, K, C3), (0, 3, 2, 1))
    return (new_xyz, feats)


# speedup vs baseline: 11.3747x; 1.0480x over previous
import functools

import jax
import jax.numpy as jnp
from jax import lax
from jax.experimental import pallas as pl
from jax.experimental.pallas import tpu as pltpu
from jax.experimental.pallas import tpu_sc as plsc

_NPOINT = 512
_NSAMPLE = 32


def _fps_body(xyz_ref, new_ref):
    x = xyz_ref[:, 0, :]
    y = xyz_ref[:, 1, :]
    z = xyz_ref[:, 2, :]
    B, N = x.shape
    S = new_ref.shape[2]
    iota = jax.lax.broadcasted_iota(jnp.int32, (B, N), 1)
    iota_s = jax.lax.broadcasted_iota(jnp.int32, (B, S), 1)

    def step(t, carry):
        dist, far, ax, ay, az = carry
        oh = iota == far
        cx = jnp.sum(jnp.where(oh, x, 0.0), axis=1, keepdims=True)
        cy = jnp.sum(jnp.where(oh, y, 0.0), axis=1, keepdims=True)
        cz = jnp.sum(jnp.where(oh, z, 0.0), axis=1, keepdims=True)
        sl = iota_s == t
        ax = jnp.where(sl, cx, ax)
        ay = jnp.where(sl, cy, ay)
        az = jnp.where(sl, cz, az)
        dx = x - cx
        dy = y - cy
        dz = z - cz
        d = (dx * dx + dz * dz) + dy * dy
        dist = jnp.minimum(dist, d)
        m = jnp.max(dist, axis=1, keepdims=True)
        far = jnp.min(jnp.where(dist == m, iota, N), axis=1, keepdims=True)
        return dist, far, ax, ay, az

    zs = jnp.zeros((B, S), jnp.float32)
    init = (jnp.full((B, N), 1e10, jnp.float32), jnp.zeros((B, 1), jnp.int32),
            zs, zs, zs)
    _, _, ax, ay, az = jax.lax.fori_loop(0, _NPOINT, step, init)
    new_ref[:, 0:1, :] = ax[:, None, :]
    new_ref[:, 1:2, :] = ay[:, None, :]
    new_ref[:, 2:3, :] = az[:, None, :]


def _knn_body(nxyz_ref, xyz_ref, idx_ref, d_ref):
    a = nxyz_ref[0]
    p = xyz_ref[0]
    TS = a.shape[1]
    N = p.shape[1]
    b = pl.program_id(0)
    at = jnp.transpose(a)
    asq = jnp.sum(at * at, axis=1, keepdims=True)
    psq = jnp.sum(p * p, axis=0, keepdims=True)
    m3 = jax.lax.dot_general(
        at, p, (((1,), (0,)), ((), ())), preferred_element_type=jnp.float32)
    d_ref[...] = (-2.0 * m3 + asq) + psq
    iota = jax.lax.broadcasted_iota(jnp.int32, (TS, N), 1)
    K = idx_ref.shape[2]
    iota_k = jax.lax.broadcasted_iota(jnp.int32, (TS, K), 1)

    def step(k, acc):
        d = d_ref[...]
        m = jnp.min(d, axis=1, keepdims=True)
        sel = jnp.min(jnp.where(d == m, iota, N), axis=1, keepdims=True)
        acc = jnp.where(iota_k == k, sel, acc)
        d_ref[...] = jnp.where(iota == sel, jnp.float32(jnp.inf), d)
        return acc

    acc = jax.lax.fori_loop(0, _NSAMPLE, step, jnp.zeros((TS, K), jnp.int32))
    idx_ref[0] = acc + b * N


def _conv1_body(x_ref, nf_ref, w_ref, b_ref, y_ref, st_ref):
    i = pl.program_id(0)
    x = x_ref[...]
    nf = nf_ref[...]
    MT = x.shape[0]
    SR = nf.shape[0]
    w3 = w_ref[0:3, :]
    t1 = jax.lax.dot_general(
        nf, w3, (((1,), (0,)), ((), ())), preferred_element_type=jnp.float32)
    t1r = jnp.broadcast_to(t1[:, None, :], (SR, MT // SR, t1.shape[1]))
    t1r = t1r.reshape(MT, t1.shape[1])
    y = jax.lax.dot_general(
        x, w_ref[...], (((1,), (0,)), ((), ())),
        preferred_element_type=jnp.float32)
    y = (y - t1r) + b_ref[...]
    y_ref[...] = y

    @pl.when(i == 0)
    def _():
        st_ref[...] = jnp.zeros_like(st_ref)

    st_ref[0:1, :] += jnp.sum(y, axis=0, keepdims=True)
    st_ref[1:2, :] += jnp.sum(y * y, axis=0, keepdims=True)


def _make_bn_conv_body(m_total):
    inv_m = 1.0 / m_total

    def body(y_ref, st_in_ref, g_ref, be_ref, w_ref, b_ref, y2_ref, st_ref):
        i = pl.program_id(0)
        y = y_ref[...]
        st = st_in_ref[...]
        mean = st[0:1, :] * inv_m
        var = st[1:2, :] * inv_m - mean * mean
        yn = (y - mean) / jnp.sqrt(var + 1e-5)
        yn = yn * g_ref[...] + be_ref[...]
        xr = jnp.maximum(yn, 0.0)
        y2 = jax.lax.dot_general(
            xr, w_ref[...], (((1,), (0,)), ((), ())),
            preferred_element_type=jnp.float32) + b_ref[...]
        y2_ref[...] = y2

        @pl.when(i == 0)
        def _():
            st_ref[...] = jnp.zeros_like(st_ref)

        st_ref[0:1, :] += jnp.sum(y2, axis=0, keepdims=True)
        st_ref[1:2, :] += jnp.sum(y2 * y2, axis=0, keepdims=True)

    return body


def _make_bn_relu_body(m_total):
    inv_m = 1.0 / m_total

    def body(y_ref, st_in_ref, g_ref, be_ref, out_ref):
        y = y_ref[...]
        st = st_in_ref[...]
        mean = st[0:1, :] * inv_m
        var = st[1:2, :] * inv_m - mean * mean
        yn = (y - mean) / jnp.sqrt(var + 1e-5)
        yn = yn * g_ref[...] + be_ref[...]
        out_ref[...] = jnp.maximum(yn, 0.0)

    return body


def kernel(xyz, points, W0, b0, g0, be0, W1, b1, g1, be1, W2, b2, g2, be2):
    B, _, N = xyz.shape
    D = points.shape[1]
    S, K = _NPOINT, _NSAMPLE
    TS = 256
    M = B * S * K
    TBL_W = 128

    new_xyz = pl.pallas_call(
        _fps_body,
        out_shape=jax.ShapeDtypeStruct((B, 3, S), jnp.float32),
    )(xyz)

    idx = pl.pallas_call(
        _knn_body,
        grid=(B, S // TS),
        in_specs=[
            pl.BlockSpec((1, 3, TS), lambda b, s: (b, 0, s)),
            pl.BlockSpec((1, 3, N), lambda b, s: (b, 0, 0)),
        ],
        out_specs=pl.BlockSpec((1, TS, K), lambda b, s: (b, s, 0)),
        out_shape=jax.ShapeDtypeStruct((B, S, K), jnp.int32),
        scratch_shapes=[pltpu.VMEM((TS, N), jnp.float32)],
    )(new_xyz, xyz)

    xyz_rows = jnp.transpose(xyz, (0, 2, 1)).reshape(B * N, 3)
    pts_rows = jnp.transpose(points, (0, 2, 1)).reshape(B * N, D)
    tbl = jnp.concatenate(
        [xyz_rows, pts_rows,
         jnp.zeros((B * N, TBL_W - 3 - D), jnp.float32)], axis=1)
    idx2d = idx.reshape(M // 128, 128)

    NW = 32
    rows_per_w = M // NW
    idx_rows_per_w = rows_per_w // 128
    mesh = plsc.VectorSubcoreMesh(core_axis_name="c", subcore_axis_name="s")

    @functools.partial(
        pl.kernel, mesh=mesh,
        out_type=jax.ShapeDtypeStruct((M, TBL_W), jnp.float32),
        scratch_types=[
            pltpu.VMEM((idx_rows_per_w, 128), jnp.int32),
            pltpu.VMEM((128, TBL_W), jnp.float32),
            pltpu.SemaphoreType.DMA,
        ],
    )
    def _gather_sc(tbl_hbm, idx_hbm, out_hbm, idx_v, rows_v, sem):
        wid = lax.axis_index("s") * 2 + lax.axis_index("c")
        pltpu.sync_copy(
            idx_hbm.at[pl.ds(wid * idx_rows_per_w, idx_rows_per_w)], idx_v)

        def body(j, _):
            pltpu.async_copy(tbl_hbm.at[idx_v.at[j]], rows_v, sem).wait()
            pltpu.sync_copy(
                rows_v, out_hbm.at[pl.ds(wid * rows_per_w + j * 128, 128)])
            return 0

        lax.fori_loop(0, idx_rows_per_w, body, 0)

    x0 = _gather_sc(tbl, idx2d)

    new_flat = jnp.transpose(new_xyz, (0, 2, 1)).reshape(B * S, 3)
    w0p = jnp.concatenate(
        [jnp.transpose(W0), jnp.zeros((TBL_W - 3 - D, W0.shape[0]),
                                      jnp.float32)], axis=0)
    MT = 4096
    grid1 = (M // MT,)
    C1 = W0.shape[0]
    y0, st0 = pl.pallas_call(
        _conv1_body,
        grid=grid1,
        in_specs=[
            pl.BlockSpec((MT, TBL_W), lambda i: (i, 0)),
            pl.BlockSpec((MT // K, 3), lambda i: (i, 0)),
            pl.BlockSpec((TBL_W, C1), lambda i: (0, 0)),
            pl.BlockSpec((1, C1), lambda i: (0, 0)),
        ],
        out_specs=[
            pl.BlockSpec((MT, C1), lambda i: (i, 0)),
            pl.BlockSpec((2, C1), lambda i: (0, 0)),
        ],
        out_shape=[
            jax.ShapeDtypeStruct((M, C1), jnp.float32),
            jax.ShapeDtypeStruct((2, C1), jnp.float32),
        ],
    )(x0, new_flat, w0p, b0[None, :])

    def bn_conv(y, st, g, be, wT, b, c_out):
        c_in = y.shape[1]
        return pl.pallas_call(
            _make_bn_conv_body(float(M)),
            grid=grid1,
            in_specs=[
                pl.BlockSpec((MT, c_in), lambda i: (i, 0)),
                pl.BlockSpec((2, c_in), lambda i: (0, 0)),
                pl.BlockSpec((1, c_in), lambda i: (0, 0)),
                pl.BlockSpec((1, c_in), lambda i: (0, 0)),
                pl.BlockSpec((c_in, c_out), lambda i: (0, 0)),
                pl.BlockSpec((1, c_out), lambda i: (0, 0)),
            ],
            out_specs=[
                pl.BlockSpec((MT, c_out), lambda i: (i, 0)),
                pl.BlockSpec((2, c_out), lambda i: (0, 0)),
            ],
            out_shape=[
                jax.ShapeDtypeStruct((M, c_out), jnp.float32),
                jax.ShapeDtypeStruct((2, c_out), jnp.float32),
            ],
        )(y, st, g[None, :], be[None, :], wT, b[None, :])

    y1, st1 = bn_conv(y0, st0, g0, be0, jnp.transpose(W1), b1, W1.shape[0])
    y2, st2 = bn_conv(y1, st1, g1, be1, jnp.transpose(W2), b2, W2.shape[0])

    C3 = W2.shape[0]
    out_flat = pl.pallas_call(
        _make_bn_relu_body(float(M)),
        grid=grid1,
        in_specs=[
            pl.BlockSpec((MT, C3), lambda i: (i, 0)),
            pl.BlockSpec((2, C3), lambda i: (0, 0)),
            pl.BlockSpec((1, C3), lambda i: (0, 0)),
            pl.BlockSpec((1, C3), lambda i: (0, 0)),
        ],
        out_specs=pl.BlockSpec((MT, C3), lambda i: (i, 0)),
        out_shape=jax.ShapeDtypeStruct((M, C3), jnp.float32),
    )(y2, st2, g2[None, :], be2[None, :])

    feats = jnp.transpose(out_flat.reshape(B, S, K, C3), (0, 3, 2, 1))
    return (new_xyz, feats)


# speedup vs baseline: 11.6058x; 1.0203x over previous
import functools

import jax
import jax.numpy as jnp
from jax import lax
from jax.experimental import pallas as pl
from jax.experimental.pallas import tpu as pltpu
from jax.experimental.pallas import tpu_sc as plsc

_NPOINT = 512
_NSAMPLE = 32


def _fps_body(xyz_ref, new_ref):
    x = xyz_ref[:, 0, :]
    y = xyz_ref[:, 1, :]
    z = xyz_ref[:, 2, :]
    B, N = x.shape
    S = new_ref.shape[2]
    iota = jax.lax.broadcasted_iota(jnp.int32, (B, N), 1)
    iota_s = jax.lax.broadcasted_iota(jnp.int32, (B, S), 1)

    def step(t, carry):
        dist, far, ax, ay, az = carry
        oh = iota == far
        cx = jnp.sum(jnp.where(oh, x, 0.0), axis=1, keepdims=True)
        cy = jnp.sum(jnp.where(oh, y, 0.0), axis=1, keepdims=True)
        cz = jnp.sum(jnp.where(oh, z, 0.0), axis=1, keepdims=True)
        sl = iota_s == t
        ax = jnp.where(sl, cx, ax)
        ay = jnp.where(sl, cy, ay)
        az = jnp.where(sl, cz, az)
        dx = x - cx
        dy = y - cy
        dz = z - cz
        d = (dx * dx + dz * dz) + dy * dy
        dist = jnp.minimum(dist, d)
        m = jnp.max(dist, axis=1, keepdims=True)
        far = jnp.min(jnp.where(dist == m, iota, N), axis=1, keepdims=True)
        return dist, far, ax, ay, az

    zs = jnp.zeros((B, S), jnp.float32)
    init = (jnp.full((B, N), 1e10, jnp.float32), jnp.zeros((B, 1), jnp.int32),
            zs, zs, zs)
    _, _, ax, ay, az = jax.lax.fori_loop(0, _NPOINT, step, init)
    new_ref[:, 0:1, :] = ax[:, None, :]
    new_ref[:, 1:2, :] = ay[:, None, :]
    new_ref[:, 2:3, :] = az[:, None, :]


def _knn_body(nxyz_ref, xyz_ref, idx_ref, d_ref):
    a = nxyz_ref[0]
    p = xyz_ref[0]
    TS = a.shape[1]
    N = p.shape[1]
    b = pl.program_id(0)
    at = jnp.transpose(a)
    asq = jnp.sum(at * at, axis=1, keepdims=True)
    psq = jnp.sum(p * p, axis=0, keepdims=True)
    m3 = jax.lax.dot_general(
        at, p, (((1,), (0,)), ((), ())), preferred_element_type=jnp.float32)
    d_ref[...] = (-2.0 * m3 + asq) + psq
    iota = jax.lax.broadcasted_iota(jnp.int32, (TS, N), 1)
    K = idx_ref.shape[2]
    iota_k = jax.lax.broadcasted_iota(jnp.int32, (TS, K), 1)

    def step(k, acc):
        d = d_ref[...]
        m = jnp.min(d, axis=1, keepdims=True)
        sel = jnp.min(jnp.where(d == m, iota, N), axis=1, keepdims=True)
        acc = jnp.where(iota_k == k, sel, acc)
        d_ref[...] = jnp.where(iota == sel, jnp.float32(jnp.inf), d)
        return acc

    acc = jax.lax.fori_loop(0, _NSAMPLE, step, jnp.zeros((TS, K), jnp.int32))
    idx_ref[0] = acc + b * N


def _conv1_body(x_ref, nf_ref, w_ref, b_ref, y_ref, st_ref):
    i = pl.program_id(0)
    x = x_ref[...]
    nf = nf_ref[...]
    MT = x.shape[0]
    SR = nf.shape[0]
    w3 = w_ref[0:3, :]
    t1 = jax.lax.dot_general(
        nf, w3, (((1,), (0,)), ((), ())), preferred_element_type=jnp.float32)
    t1r = jnp.broadcast_to(t1[:, None, :], (SR, MT // SR, t1.shape[1]))
    t1r = t1r.reshape(MT, t1.shape[1])
    y = jax.lax.dot_general(
        x, w_ref[...], (((1,), (0,)), ((), ())),
        preferred_element_type=jnp.float32)
    y = (y - t1r) + b_ref[...]
    y_ref[...] = y

    @pl.when(i == 0)
    def _():
        st_ref[...] = jnp.zeros_like(st_ref)

    st_ref[0:1, :] += jnp.sum(y, axis=0, keepdims=True)
    st_ref[1:2, :] += jnp.sum(y * y, axis=0, keepdims=True)


def _make_bn_conv_body(m_total):
    inv_m = 1.0 / m_total

    def body(y_ref, st_in_ref, g_ref, be_ref, w_ref, b_ref, y2_ref, st_ref):
        i = pl.program_id(0)
        y = y_ref[...]
        st = st_in_ref[...]
        mean = st[0:1, :] * inv_m
        var = st[1:2, :] * inv_m - mean * mean
        yn = (y - mean) / jnp.sqrt(var + 1e-5)
        yn = yn * g_ref[...] + be_ref[...]
        xr = jnp.maximum(yn, 0.0)
        y2 = jax.lax.dot_general(
            xr, w_ref[...], (((1,), (0,)), ((), ())),
            preferred_element_type=jnp.float32) + b_ref[...]
        y2_ref[...] = y2

        @pl.when(i == 0)
        def _():
            st_ref[...] = jnp.zeros_like(st_ref)

        st_ref[0:1, :] += jnp.sum(y2, axis=0, keepdims=True)
        st_ref[1:2, :] += jnp.sum(y2 * y2, axis=0, keepdims=True)

    return body


def _make_bn_relu_body(m_total):
    inv_m = 1.0 / m_total

    def body(y_ref, st_in_ref, g_ref, be_ref, out_ref):
        y = y_ref[...]
        st = st_in_ref[...]
        mean = st[0:1, :] * inv_m
        var = st[1:2, :] * inv_m - mean * mean
        yn = (y - mean) / jnp.sqrt(var + 1e-5)
        yn = yn * g_ref[...] + be_ref[...]
        out_ref[...] = jnp.maximum(yn, 0.0)

    return body


def kernel(xyz, points, W0, b0, g0, be0, W1, b1, g1, be1, W2, b2, g2, be2):
    B, _, N = xyz.shape
    D = points.shape[1]
    S, K = _NPOINT, _NSAMPLE
    TS = 512
    M = B * S * K
    TBL_W = 128

    new_xyz = pl.pallas_call(
        _fps_body,
        out_shape=jax.ShapeDtypeStruct((B, 3, S), jnp.float32),
    )(xyz)

    idx = pl.pallas_call(
        _knn_body,
        grid=(B, S // TS),
        in_specs=[
            pl.BlockSpec((1, 3, TS), lambda b, s: (b, 0, s)),
            pl.BlockSpec((1, 3, N), lambda b, s: (b, 0, 0)),
        ],
        out_specs=pl.BlockSpec((1, TS, K), lambda b, s: (b, s, 0)),
        out_shape=jax.ShapeDtypeStruct((B, S, K), jnp.int32),
        scratch_shapes=[pltpu.VMEM((TS, N), jnp.float32)],
    )(new_xyz, xyz)

    xyz_rows = jnp.transpose(xyz, (0, 2, 1)).reshape(B * N, 3)
    pts_rows = jnp.transpose(points, (0, 2, 1)).reshape(B * N, D)
    tbl = jnp.concatenate(
        [xyz_rows, pts_rows,
         jnp.zeros((B * N, TBL_W - 3 - D), jnp.float32)], axis=1)
    idx2d = idx.reshape(M // 128, 128)

    NW = 32
    rows_per_w = M // NW
    idx_rows_per_w = rows_per_w // 128
    mesh = plsc.VectorSubcoreMesh(core_axis_name="c", subcore_axis_name="s")

    @functools.partial(
        pl.kernel, mesh=mesh,
        out_type=jax.ShapeDtypeStruct((M, TBL_W), jnp.float32),
        scratch_types=[
            pltpu.VMEM((idx_rows_per_w, 128), jnp.int32),
            pltpu.VMEM((128, TBL_W), jnp.float32),
            pltpu.SemaphoreType.DMA,
        ],
    )
    def _gather_sc(tbl_hbm, idx_hbm, out_hbm, idx_v, rows_v, sem):
        wid = lax.axis_index("s") * 2 + lax.axis_index("c")
        pltpu.sync_copy(
            idx_hbm.at[pl.ds(wid * idx_rows_per_w, idx_rows_per_w)], idx_v)

        def body(j, _):
            pltpu.async_copy(tbl_hbm.at[idx_v.at[j]], rows_v, sem).wait()
            pltpu.sync_copy(
                rows_v, out_hbm.at[pl.ds(wid * rows_per_w + j * 128, 128)])
            return 0

        lax.fori_loop(0, idx_rows_per_w, body, 0)

    x0 = _gather_sc(tbl, idx2d)

    new_flat = jnp.transpose(new_xyz, (0, 2, 1)).reshape(B * S, 3)
    w0p = jnp.concatenate(
        [jnp.transpose(W0), jnp.zeros((TBL_W - 3 - D, W0.shape[0]),
                                      jnp.float32)], axis=0)
    MT = 4096
    grid1 = (M // MT,)
    C1 = W0.shape[0]
    y0, st0 = pl.pallas_call(
        _conv1_body,
        grid=grid1,
        in_specs=[
            pl.BlockSpec((MT, TBL_W), lambda i: (i, 0)),
            pl.BlockSpec((MT // K, 3), lambda i: (i, 0)),
            pl.BlockSpec((TBL_W, C1), lambda i: (0, 0)),
            pl.BlockSpec((1, C1), lambda i: (0, 0)),
        ],
        out_specs=[
            pl.BlockSpec((MT, C1), lambda i: (i, 0)),
            pl.BlockSpec((2, C1), lambda i: (0, 0)),
        ],
        out_shape=[
            jax.ShapeDtypeStruct((M, C1), jnp.float32),
            jax.ShapeDtypeStruct((2, C1), jnp.float32),
        ],
    )(x0, new_flat, w0p, b0[None, :])

    def bn_conv(y, st, g, be, wT, b, c_out):
        c_in = y.shape[1]
        return pl.pallas_call(
            _make_bn_conv_body(float(M)),
            grid=grid1,
            in_specs=[
                pl.BlockSpec((MT, c_in), lambda i: (i, 0)),
                pl.BlockSpec((2, c_in), lambda i: (0, 0)),
                pl.BlockSpec((1, c_in), lambda i: (0, 0)),
                pl.BlockSpec((1, c_in), lambda i: (0, 0)),
                pl.BlockSpec((c_in, c_out), lambda i: (0, 0)),
                pl.BlockSpec((1, c_out), lambda i: (0, 0)),
            ],
            out_specs=[
                pl.BlockSpec((MT, c_out), lambda i: (i, 0)),
                pl.BlockSpec((2, c_out), lambda i: (0, 0)),
            ],
            out_shape=[
                jax.ShapeDtypeStruct((M, c_out), jnp.float32),
                jax.ShapeDtypeStruct((2, c_out), jnp.float32),
            ],
        )(y, st, g[None, :], be[None, :], wT, b[None, :])

    y1, st1 = bn_conv(y0, st0, g0, be0, jnp.transpose(W1), b1, W1.shape[0])
    y2, st2 = bn_conv(y1, st1, g1, be1, jnp.transpose(W2), b2, W2.shape[0])

    C3 = W2.shape[0]
    out_flat = pl.pallas_call(
        _make_bn_relu_body(float(M)),
        grid=grid1,
        in_specs=[
            pl.BlockSpec((MT, C3), lambda i: (i, 0)),
            pl.BlockSpec((2, C3), lambda i: (0, 0)),
            pl.BlockSpec((1, C3), lambda i: (0, 0)),
            pl.BlockSpec((1, C3), lambda i: (0, 0)),
        ],
        out_specs=pl.BlockSpec((MT, C3), lambda i: (i, 0)),
        out_shape=jax.ShapeDtypeStruct((M, C3), jnp.float32),
    )(y2, st2, g2[None, :], be2[None, :])

    feats = jnp.transpose(out_flat.reshape(B, S, K, C3), (0, 3, 2, 1))
    return (new_xyz, feats)
